# Initial kernel scaffold; baseline (speedup 1.0000x reference)
import functools

import jax
import jax.numpy as jnp
from jax import lax
from jax.experimental import pallas as pl
from jax.experimental.pallas import tpu as pltpu
from jax.experimental.pallas import tpu_sc as plsc

N_RAYS = 16384
N_SAMPLES = 64
GRID = 64
S = GRID + 1
SH_DIM = 9
DATA_DIM = 28
DPAD = 32
NEAR = 0.0
FAR = 2.0
BG = 1.0
STEP = (FAR - NEAR) / N_SAMPLES

NC = 2
NS = 16
LANES = 16
NW = NC * NS
RAYS_PER_TILE = N_RAYS // NW
GROUPS = RAYS_PER_TILE // LANES

_OFF = [0, 1, S, S + 1, S * S, S * S + 1, S * S + S, S * S + S + 1]

_C0 = 0.28209479177387814
_C1 = 0.4886025119029199
_C20 = 1.0925484305920792
_C21 = -1.0925484305920792
_C22 = 0.31539156525252005
_C23 = -1.0925484305920792
_C24 = 0.5462742152960396


def _body(table_ref, rays_ref, out_ref,
          rayv, idxv, wbuf, rows0, rows1, outv, sem0, sem1):
  cid = lax.axis_index("c")
  sid = lax.axis_index("s")
  wid = sid * NC + cid

  pltpu.sync_copy(rays_ref.at[wid], rayv)

  lane = lax.iota(jnp.int32, LANES)
  rvecs = [cc * LANES + lane for cc in range(8)]

  def start(t, rb, sb):
    pltpu.make_async_copy(table_ref.at[idxv.at[t]], rb, sb).start()

  def wait(t, rb, sb):
    pltpu.make_async_copy(table_ref.at[idxv.at[t]], rb, sb).wait()

  def group_body(g, carry0):
    sl = pl.ds(g * LANES, LANES)
    ox = rayv[0, sl]
    oy = rayv[1, sl]
    oz = rayv[2, sl]
    dx = rayv[3, sl]
    dy = rayv[4, sl]
    dz = rayv[5, sl]
    delta = rayv[6, sl]

    b0 = jnp.full((LANES,), _C0, jnp.float32)
    b1 = -_C1 * dy
    b2 = _C1 * dz
    b3 = -_C1 * dx
    b4 = _C20 * dx * dy
    b5 = _C21 * dy * dz
    b6 = _C22 * (2.0 * dz * dz - dx * dx - dy * dy)
    b7 = _C23 * dx * dz
    b8 = _C24 * (dx * dx - dy * dy)
    bvec = [b0, b1, b2, b3, b4, b5, b6, b7, b8]

    def p1(t, carry):
      tt = (t.astype(jnp.float32) + 0.5) * STEP + NEAR
      px = ox + tt * dx
      py = oy + tt * dy
      pz = oz + tt * dz
      posx = jnp.clip(0.5 + 0.5 * px, 0.0, 1.0 - 1e-6) * GRID
      posy = jnp.clip(0.5 + 0.5 * py, 0.0, 1.0 - 1e-6) * GRID
      posz = jnp.clip(0.5 + 0.5 * pz, 0.0, 1.0 - 1e-6) * GRID
      ix = posx.astype(jnp.int32)
      iy = posy.astype(jnp.int32)
      iz = posz.astype(jnp.int32)
      fx = posx - ix.astype(jnp.float32)
      fy = posy - iy.astype(jnp.float32)
      fz = posz - iz.astype(jnp.float32)
      idx000 = (ix * S + iy) * S + iz
      for cc in range(8):
        idxv[t, pl.ds(cc * LANES, LANES)] = idx000 + _OFF[cc]
      wbuf[t, pl.ds(0, LANES)] = fx
      wbuf[t, pl.ds(LANES, LANES)] = fy
      wbuf[t, pl.ds(2 * LANES, LANES)] = fz
      return carry

    lax.fori_loop(0, N_SAMPLES, p1, 0)

    start(0, rows0, sem0)
    start(1, rows1, sem1)

    def p2(tp, carry):
      tr, aw, rr, rg, rb_ = carry
      for par, (rbuf, sbuf) in enumerate(((rows0, sem0), (rows1, sem1))):
        t = 2 * tp + par
        wait(t, rbuf, sbuf)
        fx = wbuf[t, pl.ds(0, LANES)]
        fy = wbuf[t, pl.ds(LANES, LANES)]
        fz = wbuf[t, pl.ds(2 * LANES, LANES)]
        wx0 = 1.0 - fx
        wy0 = 1.0 - fy
        wz0 = 1.0 - fz
        wxy = [wx0 * wy0, wx0 * fy, fx * wy0, fx * fy]
        w = []
        for cc in range(8):
          wz = wz0 if (cc & 1) == 0 else fz
          w.append(wxy[cc >> 1] * wz)
        acc = [jnp.zeros((LANES,), jnp.float32) for _ in range(3)]
        sig = jnp.zeros((LANES,), jnp.float32)
        for f in range(DATA_DIM):
          cvec = jnp.full((LANES,), f, jnp.int32)
          v = w[0] * plsc.load_gather(rbuf, [rvecs[0], cvec])
          for cc in range(1, 8):
            v = v + w[cc] * plsc.load_gather(rbuf, [rvecs[cc], cvec])
          if f < 3 * SH_DIM:
            acc[f // SH_DIM] = acc[f // SH_DIM] + v * bvec[f % SH_DIM]
          else:
            sig = jnp.maximum(v, 0.0)
        col = [1.0 / (1.0 + jnp.exp(-a)) for a in acc]
        alpha = 1.0 - jnp.exp(-sig * delta)
        wgt = alpha * tr
        rr = rr + wgt * col[0]
        rg = rg + wgt * col[1]
        rb_ = rb_ + wgt * col[2]
        aw = aw + wgt
        tr = tr * (1.0 - alpha + 1e-10)

        @pl.when(t + 2 < N_SAMPLES)
        def _():
          start(t + 2, rbuf, sbuf)

      return (tr, aw, rr, rg, rb_)

    ones = jnp.ones((LANES,), jnp.float32)
    zeros = jnp.zeros((LANES,), jnp.float32)
    tr, aw, rr, rg, rb_ = lax.fori_loop(
        0, N_SAMPLES // 2, p2, (ones, zeros, zeros, zeros, zeros))
    outv[0, sl] = rr + BG * (1.0 - aw)
    outv[1, sl] = rg + BG * (1.0 - aw)
    outv[2, sl] = rb_ + BG * (1.0 - aw)
    outv[3, sl] = aw
    return carry0

  lax.fori_loop(0, GROUPS, group_body, 0)
  pltpu.sync_copy(outv, out_ref.at[wid])


@jax.jit
def kernel(rays_o, rays_d, data):
  norm = jnp.linalg.norm(rays_d, axis=-1, keepdims=True)
  dn = rays_d / (norm + 1e-9)
  delta = STEP * norm
  pad = jnp.zeros((N_RAYS, 1), jnp.float32)
  rd = jnp.concatenate([rays_o, dn, delta, pad], axis=1)
  rays_packed = rd.T.reshape(8, NW, RAYS_PER_TILE).transpose(1, 0, 2)
  table = jnp.pad(data, ((0, 0), (0, DPAD - DATA_DIM)))

  mesh = plsc.VectorSubcoreMesh(
      core_axis_name="c", subcore_axis_name="s",
      num_cores=NC, num_subcores=NS)
  run = pl.kernel(
      _body,
      out_type=jax.ShapeDtypeStruct((NW, 4, RAYS_PER_TILE), jnp.float32),
      mesh=mesh,
      scratch_types=[
          pltpu.VMEM((8, RAYS_PER_TILE), jnp.float32),
          pltpu.VMEM((N_SAMPLES, 8 * LANES), jnp.int32),
          pltpu.VMEM((N_SAMPLES, 3 * LANES), jnp.float32),
          pltpu.VMEM((8 * LANES, DPAD), jnp.float32),
          pltpu.VMEM((8 * LANES, DPAD), jnp.float32),
          pltpu.VMEM((4, RAYS_PER_TILE), jnp.float32),
          pltpu.SemaphoreType.DMA,
          pltpu.SemaphoreType.DMA,
      ],
  )
  out = run(table, rays_packed)
  return out.transpose(0, 2, 1).reshape(N_RAYS, 4)[:, :3]


# speedup vs baseline: 37.4627x; 37.4627x over previous
import functools

import jax
import jax.numpy as jnp
from jax import lax
from jax.experimental import pallas as pl
from jax.experimental.pallas import tpu as pltpu
from jax.experimental.pallas import tpu_sc as plsc

N_RAYS = 16384
N_SAMPLES = 64
GRID = 64
S = GRID + 1
SH_DIM = 9
DATA_DIM = 28
DPAD = 32
NEAR = 0.0
FAR = 2.0
BG = 1.0
STEP = (FAR - NEAR) / N_SAMPLES

NC = 2
NS = 16
LANES = 16
NW = NC * NS
RAYS_PER_TILE = N_RAYS // NW
GROUPS = RAYS_PER_TILE // LANES

_OFF = [0, 1, S, S + 1, S * S, S * S + 1, S * S + S, S * S + S + 1]

_C0 = 0.28209479177387814
_C1 = 0.4886025119029199
_C20 = 1.0925484305920792
_C21 = -1.0925484305920792
_C22 = 0.31539156525252005
_C23 = -1.0925484305920792
_C24 = 0.5462742152960396


def _body(table_ref, rays_ref, out_ref,
          rayv, idxv, wbuf, rows0, rows1, outv, sem0, sem1):
  cid = lax.axis_index("c")
  sid = lax.axis_index("s")
  wid = sid * NC + cid

  pltpu.sync_copy(rays_ref.at[wid], rayv)

  lane = lax.iota(jnp.int32, LANES)
  rvecs = [cc * LANES + lane for cc in range(8)]

  def start(t, rb, sb):
    pltpu.make_async_copy(table_ref.at[idxv.at[t]], rb, sb).start()

  def wait(t, rb, sb):
    pltpu.make_async_copy(table_ref.at[idxv.at[t]], rb, sb).wait()

  def group_body(g, carry0):
    sl = pl.ds(g * LANES, LANES)
    ox = rayv[0, sl]
    oy = rayv[1, sl]
    oz = rayv[2, sl]
    dx = rayv[3, sl]
    dy = rayv[4, sl]
    dz = rayv[5, sl]
    delta = rayv[6, sl]

    b0 = jnp.full((LANES,), _C0, jnp.float32)
    b1 = -_C1 * dy
    b2 = _C1 * dz
    b3 = -_C1 * dx
    b4 = _C20 * dx * dy
    b5 = _C21 * dy * dz
    b6 = _C22 * (2.0 * dz * dz - dx * dx - dy * dy)
    b7 = _C23 * dx * dz
    b8 = _C24 * (dx * dx - dy * dy)
    bvec = [b0, b1, b2, b3, b4, b5, b6, b7, b8]

    def p1(t, carry):
      tt = (t.astype(jnp.float32) + 0.5) * STEP + NEAR
      px = ox + tt * dx
      py = oy + tt * dy
      pz = oz + tt * dz
      posx = jnp.clip(0.5 + 0.5 * px, 0.0, 1.0 - 1e-6) * GRID
      posy = jnp.clip(0.5 + 0.5 * py, 0.0, 1.0 - 1e-6) * GRID
      posz = jnp.clip(0.5 + 0.5 * pz, 0.0, 1.0 - 1e-6) * GRID
      ix = posx.astype(jnp.int32)
      iy = posy.astype(jnp.int32)
      iz = posz.astype(jnp.int32)
      fx = posx - ix.astype(jnp.float32)
      fy = posy - iy.astype(jnp.float32)
      fz = posz - iz.astype(jnp.float32)
      idx000 = (ix * S + iy) * S + iz
      for cc in range(8):
        idxv[t, pl.ds(cc * LANES, LANES)] = idx000 + _OFF[cc]
      wbuf[t, pl.ds(0, LANES)] = fx
      wbuf[t, pl.ds(LANES, LANES)] = fy
      wbuf[t, pl.ds(2 * LANES, LANES)] = fz
      return carry

    lax.fori_loop(0, N_SAMPLES, p1, 0)

    start(0, rows0, sem0)
    start(1, rows1, sem1)

    def p2(tp, carry):
      tr, aw, rr, rg, rb_ = carry
      for par, (rbuf, sbuf) in enumerate(((rows0, sem0), (rows1, sem1))):
        t = 2 * tp + par
        wait(t, rbuf, sbuf)
        fx = wbuf[t, pl.ds(0, LANES)]
        fy = wbuf[t, pl.ds(LANES, LANES)]
        fz = wbuf[t, pl.ds(2 * LANES, LANES)]
        wx0 = 1.0 - fx
        wy0 = 1.0 - fy
        wz0 = 1.0 - fz
        wxy = [wx0 * wy0, wx0 * fy, fx * wy0, fx * fy]
        w = []
        for cc in range(8):
          wz = wz0 if (cc & 1) == 0 else fz
          w.append(wxy[cc >> 1] * wz)
        acc = [jnp.zeros((LANES,), jnp.float32) for _ in range(3)]
        sig = jnp.zeros((LANES,), jnp.float32)
        for f in range(DATA_DIM):
          cvec = jnp.full((LANES,), f, jnp.int32)
          v = w[0] * plsc.load_gather(rbuf, [rvecs[0], cvec])
          for cc in range(1, 8):
            v = v + w[cc] * plsc.load_gather(rbuf, [rvecs[cc], cvec])
          if f < 3 * SH_DIM:
            acc[f // SH_DIM] = acc[f // SH_DIM] + v * bvec[f % SH_DIM]
          else:
            sig = jnp.maximum(v, 0.0)
        col = [1.0 / (1.0 + jnp.exp(-a)) for a in acc]
        alpha = 1.0 - jnp.exp(-sig * delta)
        wgt = alpha * tr
        rr = rr + wgt * col[0]
        rg = rg + wgt * col[1]
        rb_ = rb_ + wgt * col[2]
        aw = aw + wgt
        tr = tr * (1.0 - alpha + 1e-10)

        @pl.when(t + 2 < N_SAMPLES)
        def _():
          start(t + 2, rbuf, sbuf)

      return (tr, aw, rr, rg, rb_)

    ones = jnp.ones((LANES,), jnp.float32)
    zeros = jnp.zeros((LANES,), jnp.float32)
    tr, aw, rr, rg, rb_ = lax.fori_loop(
        0, N_SAMPLES // 2, p2, (ones, zeros, zeros, zeros, zeros))
    outv[0, sl] = rr + BG * (1.0 - aw)
    outv[1, sl] = rg + BG * (1.0 - aw)
    outv[2, sl] = rb_ + BG * (1.0 - aw)
    outv[3, sl] = aw
    return carry0

  lax.fori_loop(0, GROUPS, group_body, 0)
  pltpu.sync_copy(outv, out_ref.at[wid])


@jax.jit
def kernel(rays_o, rays_d, data):
  norm = jnp.linalg.norm(rays_d, axis=-1, keepdims=True)
  dn = rays_d / (norm + 1e-9)
  delta = STEP * norm
  pad = jnp.zeros((N_RAYS, 1), jnp.float32)
  rd = jnp.concatenate([rays_o, dn, delta, pad], axis=1)
  rays_packed = rd.T.reshape(8, NW, RAYS_PER_TILE).transpose(1, 0, 2)
  table = jnp.pad(data, ((0, 0), (0, DPAD - DATA_DIM)))

  mesh = plsc.VectorSubcoreMesh(
      core_axis_name="c", subcore_axis_name="s",
      num_cores=NC, num_subcores=NS)
  run = pl.kernel(
      _body,
      out_type=jax.ShapeDtypeStruct((NW, 4, RAYS_PER_TILE), jnp.float32),
      mesh=mesh,
      scratch_types=[
          pltpu.VMEM((8, RAYS_PER_TILE), jnp.float32),
          pltpu.VMEM((N_SAMPLES, 8 * LANES), jnp.int32),
          pltpu.VMEM((N_SAMPLES, 3 * LANES), jnp.float32),
          pltpu.VMEM((8 * LANES, DPAD), jnp.float32),
          pltpu.VMEM((8 * LANES, DPAD), jnp.float32),
          pltpu.VMEM((4, RAYS_PER_TILE), jnp.float32),
          pltpu.SemaphoreType.DMA,
          pltpu.SemaphoreType.DMA,
      ],
      compiler_params=pltpu.CompilerParams(
          needs_layout_passes=False, use_tc_tiling_on_sc=False),
  )
  out = run(table, rays_packed)
  return out.transpose(0, 2, 1).reshape(N_RAYS, 4)[:, :3]
